# baseline (speedup 1.0000x reference)
import functools
import math

import numpy as np
import jax
import jax.numpy as jnp
from jax import lax
from jax.experimental import pallas as pl
from jax.experimental.pallas import tpu as pltpu
from jax.experimental.pallas import tpu_sc as plsc

K = 10
N = 10000
E = 320000
F = 128

NC = 2
NS = 16
NW = NC * NS
KB = 128
CH = 80
EPW = CH * KB
EPAD = NW * EPW
NB = 2
NR = CH // 2
PH = 1
FH = F // PH

NPAD = 10240
RPW = NPAD // NS
DW = 16
BLK = 640
GRID = NPAD // BLK


def _cheb_scalar(i, x):
    if i == 0:
        return 1.0
    if i == 1:
        return x
    t0, t1 = 1.0, x
    for _ in range(2, i + 1):
        t0, t1 = t1, 2.0 * x * t1 - t0
    return t1


def _coe_rows():
    rows = np.zeros((K + 1, K + 1), dtype=np.float32)
    for i in range(K + 1):
        for j in range(K + 1):
            xj = math.cos((K - j + 0.5) * math.pi / (K + 1))
            rows[i, j] = _cheb_scalar(i, xj) * (2.0 / (K + 1))
    return rows


_CROWS = _coe_rows()


def _coe_dot(i, temp_ref):
    ci = jnp.float32(0.0)
    for j in range(K + 1):
        ci = ci + float(_CROWS[i, j]) * temp_ref[0, j]
    return ci


_sc_mesh = plsc.VectorSubcoreMesh(core_axis_name="c", subcore_axis_name="s")


@functools.partial(
    pl.kernel,
    out_type=jax.ShapeDtypeStruct((NC, PH, NPAD, FH), jnp.float32),
    mesh=_sc_mesh,
    scratch_types=[
        pltpu.VMEM((CH, KB), jnp.int32),
        pltpu.VMEM((CH, KB), jnp.int32),
        pltpu.VMEM((KB, FH), jnp.float32),
        pltpu.VMEM_SHARED((NPAD, FH), jnp.float32),
        pltpu.SemaphoreType.DMA,
    ],
)
def _sc_prop(table_hbm, srcs_hbm, dsts_hbm, zeros_hbm, out_hbm,
             src_v, dst_v, rows, acc, sem):
    c = lax.axis_index("c")
    s = lax.axis_index("s")
    w = s * NC + c
    base = s * RPW
    stripe = pl.ds(base, RPW)

    pltpu.sync_copy(zeros_hbm, rows)
    for k in range(RPW // KB):
        pltpu.async_copy(rows, acc.at[pl.ds(base + k * KB, KB)], sem)
    pltpu.sync_copy(srcs_hbm.at[w], src_v)
    pltpu.sync_copy(dsts_hbm.at[w], dst_v)
    for k in range(RPW // KB):
        pltpu.make_async_copy(rows, acc.at[pl.ds(base + k * KB, KB)],
                              sem).wait()
    plsc.subcore_barrier()

    for p in range(PH):
        tab = table_hbm.at[p]

        def _chunk(j, carry):
            pltpu.async_copy(tab.at[src_v.at[j]], rows, sem).wait()
            pltpu.sync_copy(rows, acc.at[dst_v.at[j]], add=True)
            return carry

        lax.fori_loop(0, CH, _chunk, 0)
        plsc.subcore_barrier()

        pltpu.sync_copy(acc.at[stripe], out_hbm.at[c].at[p].at[stripe])


@functools.partial(
    pl.kernel,
    out_type=jax.ShapeDtypeStruct((NC, NPAD, DW), jnp.float32),
    mesh=_sc_mesh,
    scratch_types=[
        pltpu.VMEM((CH, KB), jnp.int32),
        pltpu.VMEM((KB, DW), jnp.float32),
        pltpu.VMEM((KB, DW), jnp.float32),
        pltpu.VMEM_SHARED((NPAD, DW), jnp.float32),
        pltpu.SemaphoreType.DMA,
    ],
    compiler_params=pltpu.CompilerParams(use_tc_tiling_on_sc=False),
)
def _sc_deg(srcs_hbm, ones_hbm, zeros_hbm, out_hbm, src_v, ones_v, zeros_v,
            acc, sem):
    c = lax.axis_index("c")
    s = lax.axis_index("s")
    w = s * NC + c
    base = s * RPW
    stripe = pl.ds(base, RPW)

    pltpu.sync_copy(zeros_hbm, zeros_v)
    off = 0
    while off < RPW:
        sz = min(KB, RPW - off)
        pltpu.sync_copy(zeros_v.at[pl.ds(0, sz)],
                        acc.at[pl.ds(base + off, sz)])
        off += sz
    pltpu.sync_copy(srcs_hbm.at[w], src_v)
    pltpu.sync_copy(ones_hbm, ones_v)
    plsc.subcore_barrier()

    def _fire(j, carry):
        pltpu.async_copy(ones_v, acc.at[src_v.at[j]], sem, add=True)
        return carry

    lax.fori_loop(0, CH, _fire, 0)

    def _drain(j, carry):
        pltpu.make_async_copy(ones_v, acc.at[src_v.at[j]], sem).wait()
        return carry

    lax.fori_loop(0, CH, _drain, 0)
    plsc.subcore_barrier()

    pltpu.sync_copy(acc.at[stripe], out_hbm.at[c].at[stripe])


def _mlp_body(x_ref, w1_ref, b1_ref, w2_ref, b2_ref, deg2_ref, temp_ref,
              h_ref, u_ref, out0_ref, dis_ref):
    b = pl.program_id(0)
    xb = x_ref[...]
    hb = lax.dot_general(xb, w1_ref[...], (((1,), (1,)), ((), ())),
                         preferred_element_type=jnp.float32) + b1_ref[...]
    hb = jnp.maximum(hb, 0.0)
    hb = lax.dot_general(hb, w2_ref[...], (((1,), (1,)), ((), ())),
                         preferred_element_type=jnp.float32) + b2_ref[...]
    rowid = b * BLK + lax.broadcasted_iota(jnp.int32, (BLK, 1), 0)
    hb = jnp.where(rowid < N, hb, 0.0)
    deg = deg2_ref[0, :, 0:1] + deg2_ref[1, :, 0:1]
    pos = deg > 0.0
    dis = jnp.where(pos, lax.rsqrt(jnp.where(pos, deg, 1.0)), 0.0)
    coe0 = _coe_dot(0, temp_ref)
    ub = dis * hb
    h_ref[...] = hb
    for p in range(PH):
        u_ref[p] = ub[:, p * FH:(p + 1) * FH]
    out0_ref[...] = (0.5 * coe0) * hb
    dis_ref[...] = dis


_mlp = pl.pallas_call(
    _mlp_body,
    grid=(GRID,),
    in_specs=[
        pl.BlockSpec((BLK, F), lambda b: (b, 0)),
        pl.BlockSpec((F, F), lambda b: (0, 0)),
        pl.BlockSpec((1, F), lambda b: (0, 0)),
        pl.BlockSpec((F, F), lambda b: (0, 0)),
        pl.BlockSpec((1, F), lambda b: (0, 0)),
        pl.BlockSpec((NC, BLK, DW), lambda b: (0, b, 0)),
        pl.BlockSpec((1, 128), lambda b: (0, 0)),
    ],
    out_specs=[
        pl.BlockSpec((BLK, F), lambda b: (b, 0)),
        pl.BlockSpec((PH, BLK, FH), lambda b: (0, b, 0)),
        pl.BlockSpec((BLK, F), lambda b: (b, 0)),
        pl.BlockSpec((BLK, 1), lambda b: (b, 0)),
    ],
    out_shape=[
        jax.ShapeDtypeStruct((NPAD, F), jnp.float32),
        jax.ShapeDtypeStruct((PH, NPAD, FH), jnp.float32),
        jax.ShapeDtypeStruct((NPAD, F), jnp.float32),
        jax.ShapeDtypeStruct((NPAD, 1), jnp.float32),
    ],
)


def _step_body(s_ref, dis_ref, tp_ref, oin_ref, temp_ref,
               tx_ref, out_ref, u_ref, *, alpha, beta, i):
    parts = [s_ref[0, p] + s_ref[1, p] for p in range(PH)]
    sblk = parts[0] if PH == 1 else jnp.concatenate(parts, axis=1)
    d = dis_ref[...]
    ci = _coe_dot(i, temp_ref)
    t = alpha * (d * sblk) + beta * tp_ref[...]
    ut = d * t
    tx_ref[...] = t
    out_ref[...] = oin_ref[...] + ci * t
    for p in range(PH):
        u_ref[p] = ut[:, p * FH:(p + 1) * FH]


def _make_step(i):
    alpha = -1.0 if i == 1 else -2.0
    beta = 0.0 if i == 1 else -1.0
    return pl.pallas_call(
        functools.partial(_step_body, alpha=alpha, beta=beta, i=i),
        grid=(GRID,),
        in_specs=[
            pl.BlockSpec((NC, PH, BLK, FH), lambda b: (0, 0, b, 0)),
            pl.BlockSpec((BLK, 1), lambda b: (b, 0)),
            pl.BlockSpec((BLK, F), lambda b: (b, 0)),
            pl.BlockSpec((BLK, F), lambda b: (b, 0)),
            pl.BlockSpec((1, 128), lambda b: (0, 0)),
        ],
        out_specs=[
            pl.BlockSpec((BLK, F), lambda b: (b, 0)),
            pl.BlockSpec((BLK, F), lambda b: (b, 0)),
            pl.BlockSpec((PH, BLK, FH), lambda b: (0, b, 0)),
        ],
        out_shape=[
            jax.ShapeDtypeStruct((NPAD, F), jnp.float32),
            jax.ShapeDtypeStruct((NPAD, F), jnp.float32),
            jax.ShapeDtypeStruct((PH, NPAD, FH), jnp.float32),
        ],
    )


_steps = {i: _make_step(i) for i in range(1, K + 1)}


def kernel(x, edge_index, edge_att, W1, b1, W2, b2, temp):
    del edge_att
    row = edge_index[0]
    col = edge_index[1]
    fill = jnp.full((EPAD - E,), N, jnp.int32)
    srcs = jnp.concatenate([row, fill]).reshape(NW, CH, KB)
    dsts = jnp.concatenate([col, fill]).reshape(NW, CH, KB)
    x_pad = jnp.pad(x, ((0, NPAD - N), (0, 0)))
    temp_pad = jnp.zeros((1, 128), jnp.float32).at[0, : K + 1].set(temp)
    zrows = jnp.zeros((KB, FH), jnp.float32)
    zrows_d = jnp.zeros((KB, DW), jnp.float32)
    ones_d = jnp.ones((KB, DW), jnp.float32)

    deg2 = _sc_deg(srcs, ones_d, zrows_d)
    h, u, out, dis = _mlp(x_pad, W1, b1.reshape(1, F), W2, b2.reshape(1, F),
                          deg2, temp_pad)
    tx0 = h
    s_p = _sc_prop(u, srcs, dsts, zrows)
    tx1, out, u = _steps[1](s_p, dis, tx0, out, temp_pad)
    for i in range(2, K + 1):
        s_p = _sc_prop(u, srcs, dsts, zrows)
        tx2, out, u = _steps[i](s_p, dis, tx1, out, temp_pad)
        tx0, tx1 = tx1, tx2
    return out[:N]


# speedup vs baseline: 2.7820x; 2.7820x over previous
import functools
import math

import numpy as np
import jax
import jax.numpy as jnp
from jax import lax
from jax.experimental import pallas as pl
from jax.experimental.pallas import tpu as pltpu
from jax.experimental.pallas import tpu_sc as plsc

K = 10
N = 10000
E = 320000
F = 128

NC = 2
NS = 16
NW = NC * NS
KB = 128
CH = 80
EPW = CH * KB
EPAD = NW * EPW
NB = 2
NR = CH // 2
PH = 1
FH = F // PH

NPAD = 10240
RPW = NPAD // NS
DW = 16
BLK = 640
GRID = NPAD // BLK


def _cheb_scalar(i, x):
    if i == 0:
        return 1.0
    if i == 1:
        return x
    t0, t1 = 1.0, x
    for _ in range(2, i + 1):
        t0, t1 = t1, 2.0 * x * t1 - t0
    return t1


def _coe_rows():
    rows = np.zeros((K + 1, K + 1), dtype=np.float32)
    for i in range(K + 1):
        for j in range(K + 1):
            xj = math.cos((K - j + 0.5) * math.pi / (K + 1))
            rows[i, j] = _cheb_scalar(i, xj) * (2.0 / (K + 1))
    return rows


_CROWS = _coe_rows()


def _coe_dot(i, temp_ref):
    ci = jnp.float32(0.0)
    for j in range(K + 1):
        ci = ci + float(_CROWS[i, j]) * temp_ref[0, j]
    return ci


_sc_mesh = plsc.VectorSubcoreMesh(core_axis_name="c", subcore_axis_name="s")


@functools.partial(
    pl.kernel,
    out_type=jax.ShapeDtypeStruct((NC, PH, NPAD, FH), jnp.float32),
    mesh=_sc_mesh,
    scratch_types=[
        pltpu.VMEM((CH, KB), jnp.int32),
        pltpu.VMEM((CH, KB), jnp.int32),
        pltpu.VMEM((KB, FH), jnp.float32),
        pltpu.VMEM_SHARED((NPAD, FH), jnp.float32),
        pltpu.SemaphoreType.DMA,
    ],
)
def _sc_prop(table_hbm, srcs_hbm, dsts_hbm, zeros_hbm, out_hbm,
             src_v, dst_v, rows, acc, sem):
    c = lax.axis_index("c")
    s = lax.axis_index("s")
    w = s * NC + c
    base = s * RPW
    stripe = pl.ds(base, RPW)

    pltpu.sync_copy(zeros_hbm, rows)
    for k in range(RPW // KB):
        pltpu.async_copy(rows, acc.at[pl.ds(base + k * KB, KB)], sem)
    pltpu.sync_copy(srcs_hbm.at[w], src_v)
    pltpu.sync_copy(dsts_hbm.at[w], dst_v)
    for k in range(RPW // KB):
        pltpu.make_async_copy(rows, acc.at[pl.ds(base + k * KB, KB)],
                              sem).wait()
    plsc.subcore_barrier()

    for p in range(PH):
        tab = table_hbm.at[p]

        def _chunk(j, carry):
            pltpu.async_copy(tab.at[src_v.at[j]], rows, sem).wait()
            pltpu.sync_copy(rows, acc.at[dst_v.at[j]], add=True)
            return carry

        lax.fori_loop(0, CH, _chunk, 0)
        plsc.subcore_barrier()

        pltpu.sync_copy(acc.at[stripe], out_hbm.at[c].at[p].at[stripe])


@functools.partial(
    pl.kernel,
    out_type=jax.ShapeDtypeStruct((NC, NPAD, DW), jnp.float32),
    mesh=_sc_mesh,
    scratch_types=[
        pltpu.VMEM((CH, KB), jnp.int32),
        pltpu.VMEM((KB, DW), jnp.float32),
        pltpu.VMEM((KB, DW), jnp.float32),
        pltpu.VMEM_SHARED((NPAD, DW), jnp.float32),
        pltpu.SemaphoreType.DMA,
    ],
    compiler_params=pltpu.CompilerParams(use_tc_tiling_on_sc=False),
)
def _sc_deg(srcs_hbm, ones_hbm, zeros_hbm, out_hbm, src_v, ones_v, zeros_v,
            acc, sem):
    c = lax.axis_index("c")
    s = lax.axis_index("s")
    w = s * NC + c
    base = s * RPW
    stripe = pl.ds(base, RPW)

    pltpu.sync_copy(zeros_hbm, zeros_v)
    off = 0
    while off < RPW:
        sz = min(KB, RPW - off)
        pltpu.sync_copy(zeros_v.at[pl.ds(0, sz)],
                        acc.at[pl.ds(base + off, sz)])
        off += sz
    pltpu.sync_copy(srcs_hbm.at[w], src_v)
    pltpu.sync_copy(ones_hbm, ones_v)
    plsc.subcore_barrier()

    def _fire(j, carry):
        pltpu.async_copy(ones_v, acc.at[src_v.at[j]], sem, add=True)
        return carry

    lax.fori_loop(0, CH, _fire, 0)

    def _drain(j, carry):
        pltpu.make_async_copy(ones_v, acc.at[src_v.at[j]], sem).wait()
        return carry

    lax.fori_loop(0, CH, _drain, 0)
    plsc.subcore_barrier()

    pltpu.sync_copy(acc.at[stripe], out_hbm.at[c].at[stripe])


def _mlp_body(x_ref, w1_ref, b1_ref, w2_ref, b2_ref, deg2_ref, temp_ref,
              h_ref, u_ref, out0_ref, dis_ref):
    b = pl.program_id(0)
    xb = x_ref[...]
    hb = lax.dot_general(xb, w1_ref[...], (((1,), (1,)), ((), ())),
                         preferred_element_type=jnp.float32) + b1_ref[...]
    hb = jnp.maximum(hb, 0.0)
    hb = lax.dot_general(hb, w2_ref[...], (((1,), (1,)), ((), ())),
                         preferred_element_type=jnp.float32) + b2_ref[...]
    rowid = b * BLK + lax.broadcasted_iota(jnp.int32, (BLK, 1), 0)
    hb = jnp.where(rowid < N, hb, 0.0)
    deg = deg2_ref[0, :, 0:1] + deg2_ref[1, :, 0:1]
    pos = deg > 0.0
    dis = jnp.where(pos, lax.rsqrt(jnp.where(pos, deg, 1.0)), 0.0)
    coe0 = _coe_dot(0, temp_ref)
    ub = dis * hb
    h_ref[...] = hb
    for p in range(PH):
        u_ref[p] = ub[:, p * FH:(p + 1) * FH]
    out0_ref[...] = (0.5 * coe0) * hb
    dis_ref[...] = dis


_mlp = pl.pallas_call(
    _mlp_body,
    grid=(GRID,),
    in_specs=[
        pl.BlockSpec((BLK, F), lambda b: (b, 0)),
        pl.BlockSpec((F, F), lambda b: (0, 0)),
        pl.BlockSpec((1, F), lambda b: (0, 0)),
        pl.BlockSpec((F, F), lambda b: (0, 0)),
        pl.BlockSpec((1, F), lambda b: (0, 0)),
        pl.BlockSpec((NC, BLK, DW), lambda b: (0, b, 0)),
        pl.BlockSpec((1, 128), lambda b: (0, 0)),
    ],
    out_specs=[
        pl.BlockSpec((BLK, F), lambda b: (b, 0)),
        pl.BlockSpec((PH, BLK, FH), lambda b: (0, b, 0)),
        pl.BlockSpec((BLK, F), lambda b: (b, 0)),
        pl.BlockSpec((BLK, 1), lambda b: (b, 0)),
    ],
    out_shape=[
        jax.ShapeDtypeStruct((NPAD, F), jnp.float32),
        jax.ShapeDtypeStruct((PH, NPAD, FH), jnp.float32),
        jax.ShapeDtypeStruct((NPAD, F), jnp.float32),
        jax.ShapeDtypeStruct((NPAD, 1), jnp.float32),
    ],
)


def _step_body(s_ref, dis_ref, tp_ref, oin_ref, temp_ref,
               tx_ref, out_ref, u_ref, *, alpha, beta, i):
    parts = [s_ref[0, p] + s_ref[1, p] for p in range(PH)]
    sblk = parts[0] if PH == 1 else jnp.concatenate(parts, axis=1)
    d = dis_ref[...]
    ci = _coe_dot(i, temp_ref)
    t = alpha * (d * sblk) + beta * tp_ref[...]
    ut = d * t
    tx_ref[...] = t
    out_ref[...] = oin_ref[...] + ci * t
    for p in range(PH):
        u_ref[p] = ut[:, p * FH:(p + 1) * FH]


def _make_step(i):
    alpha = -1.0 if i == 1 else -2.0
    beta = 0.0 if i == 1 else -1.0
    return pl.pallas_call(
        functools.partial(_step_body, alpha=alpha, beta=beta, i=i),
        grid=(GRID,),
        in_specs=[
            pl.BlockSpec((NC, PH, BLK, FH), lambda b: (0, 0, b, 0)),
            pl.BlockSpec((BLK, 1), lambda b: (b, 0)),
            pl.BlockSpec((BLK, F), lambda b: (b, 0)),
            pl.BlockSpec((BLK, F), lambda b: (b, 0)),
            pl.BlockSpec((1, 128), lambda b: (0, 0)),
        ],
        out_specs=[
            pl.BlockSpec((BLK, F), lambda b: (b, 0)),
            pl.BlockSpec((BLK, F), lambda b: (b, 0)),
            pl.BlockSpec((PH, BLK, FH), lambda b: (0, b, 0)),
        ],
        out_shape=[
            jax.ShapeDtypeStruct((NPAD, F), jnp.float32),
            jax.ShapeDtypeStruct((NPAD, F), jnp.float32),
            jax.ShapeDtypeStruct((PH, NPAD, FH), jnp.float32),
        ],
    )


_steps = {i: _make_step(i) for i in range(1, K + 1)}


def kernel(x, edge_index, edge_att, W1, b1, W2, b2, temp):
    del edge_att
    row = edge_index[0]
    col = edge_index[1]
    fill = N + jnp.arange(EPAD - E, dtype=jnp.int32) % (NPAD - N)
    srcs = jnp.concatenate([row, fill]).reshape(NW, CH, KB)
    dsts = jnp.concatenate([col, fill]).reshape(NW, CH, KB)
    x_pad = jnp.pad(x, ((0, NPAD - N), (0, 0)))
    temp_pad = jnp.zeros((1, 128), jnp.float32).at[0, : K + 1].set(temp)
    zrows = jnp.zeros((KB, FH), jnp.float32)
    zrows_d = jnp.zeros((KB, DW), jnp.float32)
    ones_d = jnp.ones((KB, DW), jnp.float32)

    deg2 = _sc_deg(srcs, ones_d, zrows_d)
    h, u, out, dis = _mlp(x_pad, W1, b1.reshape(1, F), W2, b2.reshape(1, F),
                          deg2, temp_pad)
    tx0 = h
    s_p = _sc_prop(u, srcs, dsts, zrows)
    tx1, out, u = _steps[1](s_p, dis, tx0, out, temp_pad)
    for i in range(2, K + 1):
        s_p = _sc_prop(u, srcs, dsts, zrows)
        tx2, out, u = _steps[i](s_p, dis, tx1, out, temp_pad)
        tx0, tx1 = tx1, tx2
    return out[:N]


# speedup vs baseline: 3.5230x; 1.2664x over previous
import functools
import math

import numpy as np
import jax
import jax.numpy as jnp
from jax import lax
from jax.experimental import pallas as pl
from jax.experimental.pallas import tpu as pltpu
from jax.experimental.pallas import tpu_sc as plsc

K = 10
N = 10000
E = 320000
F = 128

NC = 2
NS = 16
NW = NC * NS
KB = 128
CH = 80
EPW = CH * KB
EPAD = NW * EPW
NB = 2
NR = CH // 2
PH = 1
FH = F // PH

NPAD = 10240
RPW = NPAD // NS
DW = 16
BLK = 640
GRID = NPAD // BLK


def _cheb_scalar(i, x):
    if i == 0:
        return 1.0
    if i == 1:
        return x
    t0, t1 = 1.0, x
    for _ in range(2, i + 1):
        t0, t1 = t1, 2.0 * x * t1 - t0
    return t1


def _coe_rows():
    rows = np.zeros((K + 1, K + 1), dtype=np.float32)
    for i in range(K + 1):
        for j in range(K + 1):
            xj = math.cos((K - j + 0.5) * math.pi / (K + 1))
            rows[i, j] = _cheb_scalar(i, xj) * (2.0 / (K + 1))
    return rows


_CROWS = _coe_rows()


def _coe_dot(i, temp_ref):
    ci = jnp.float32(0.0)
    for j in range(K + 1):
        ci = ci + float(_CROWS[i, j]) * temp_ref[0, j]
    return ci


_sc_mesh = plsc.VectorSubcoreMesh(core_axis_name="c", subcore_axis_name="s")


@functools.partial(
    pl.kernel,
    out_type=jax.ShapeDtypeStruct((NC, PH, NPAD, FH), jnp.float32),
    mesh=_sc_mesh,
    scratch_types=[
        pltpu.VMEM((CH, KB), jnp.int32),
        [pltpu.VMEM((KB, FH), jnp.float32) for _ in range(2)],
        [pltpu.VMEM((KB,), jnp.int32) for _ in range(2)],
        [pltpu.VMEM((KB,), jnp.int32) for _ in range(2)],
        pltpu.VMEM_SHARED((NPAD, FH), jnp.float32),
        [pltpu.SemaphoreType.DMA for _ in range(2)],
        [pltpu.SemaphoreType.DMA for _ in range(2)],
    ],
)
def _sc_prop(table_hbm, packed_hbm, zeros_hbm, out_hbm,
             pk_v, rows, sidx, didx, acc, sem_g, sem_s):
    c = lax.axis_index("c")
    s = lax.axis_index("s")
    w = s * NC + c
    base = s * RPW
    stripe = pl.ds(base, RPW)

    pltpu.sync_copy(packed_hbm.at[w], pk_v)

    def _unpack(b, j):
        for q in range(KB // 16):
            v = pk_v[j, pl.ds(q * 16, 16)]
            sidx[b][pl.ds(q * 16, 16)] = jnp.bitwise_and(v, 0xFFFF)
            didx[b][pl.ds(q * 16, 16)] = lax.shift_right_logical(v, 16)

    pltpu.sync_copy(zeros_hbm, rows[0])
    for k in range(RPW // KB):
        pltpu.async_copy(rows[0], acc.at[pl.ds(base + k * KB, KB)], sem_s[0])
    for k in range(RPW // KB):
        pltpu.make_async_copy(rows[0], acc.at[pl.ds(base + k * KB, KB)],
                              sem_s[0]).wait()
    plsc.subcore_barrier()

    for p in range(PH):
        tab = table_hbm.at[p]

        def _g_start(b, j):
            _unpack(b, j)
            pltpu.async_copy(tab.at[sidx[b]], rows[b], sem_g[b])

        def _g_wait(b):
            pltpu.make_async_copy(tab.at[sidx[b]], rows[b], sem_g[b]).wait()

        def _s_start(b):
            pltpu.async_copy(rows[b], acc.at[didx[b]], sem_s[b], add=True)

        def _s_wait(b):
            pltpu.make_async_copy(rows[b], acc.at[didx[b]], sem_s[b]).wait()

        def _pingpong(r, first, last):
            j0 = 2 * r
            j1 = j0 + 1
            _g_wait(0)
            _s_start(0)
            if not first:
                _s_wait(1)
            _g_start(1, j1)
            _g_wait(1)
            _s_start(1)
            _s_wait(0)
            if not last:
                _g_start(0, j0 + 2)
            else:
                _s_wait(1)

        _g_start(0, 0)
        _pingpong(0, True, NR == 1)
        if NR > 2:
            def _mid(r, carry):
                _pingpong(r, False, False)
                return carry
            lax.fori_loop(1, NR - 1, _mid, 0)
        if NR > 1:
            _pingpong(NR - 1, False, True)
        plsc.subcore_barrier()

        pltpu.sync_copy(acc.at[stripe], out_hbm.at[c].at[p].at[stripe])


@functools.partial(
    pl.kernel,
    out_type=jax.ShapeDtypeStruct((NC, NPAD, DW), jnp.float32),
    mesh=_sc_mesh,
    scratch_types=[
        pltpu.VMEM((CH, KB), jnp.int32),
        pltpu.VMEM((KB, DW), jnp.float32),
        pltpu.VMEM((KB, DW), jnp.float32),
        pltpu.VMEM_SHARED((NPAD, DW), jnp.float32),
        pltpu.SemaphoreType.DMA,
    ],
    compiler_params=pltpu.CompilerParams(use_tc_tiling_on_sc=False),
)
def _sc_deg(srcs_hbm, ones_hbm, zeros_hbm, out_hbm, src_v, ones_v, zeros_v,
            acc, sem):
    c = lax.axis_index("c")
    s = lax.axis_index("s")
    w = s * NC + c
    base = s * RPW
    stripe = pl.ds(base, RPW)

    pltpu.sync_copy(zeros_hbm, zeros_v)
    off = 0
    while off < RPW:
        sz = min(KB, RPW - off)
        pltpu.sync_copy(zeros_v.at[pl.ds(0, sz)],
                        acc.at[pl.ds(base + off, sz)])
        off += sz
    pltpu.sync_copy(srcs_hbm.at[w], src_v)
    pltpu.sync_copy(ones_hbm, ones_v)
    plsc.subcore_barrier()

    def _fire(j, carry):
        pltpu.async_copy(ones_v, acc.at[src_v.at[j]], sem, add=True)
        return carry

    lax.fori_loop(0, CH, _fire, 0)

    def _drain(j, carry):
        pltpu.make_async_copy(ones_v, acc.at[src_v.at[j]], sem).wait()
        return carry

    lax.fori_loop(0, CH, _drain, 0)
    plsc.subcore_barrier()

    pltpu.sync_copy(acc.at[stripe], out_hbm.at[c].at[stripe])


def _mlp_body(x_ref, w1_ref, b1_ref, w2_ref, b2_ref, deg2_ref, temp_ref,
              h_ref, u_ref, out0_ref, dis_ref):
    b = pl.program_id(0)
    xb = x_ref[...]
    hb = lax.dot_general(xb, w1_ref[...], (((1,), (1,)), ((), ())),
                         preferred_element_type=jnp.float32) + b1_ref[...]
    hb = jnp.maximum(hb, 0.0)
    hb = lax.dot_general(hb, w2_ref[...], (((1,), (1,)), ((), ())),
                         preferred_element_type=jnp.float32) + b2_ref[...]
    rowid = b * BLK + lax.broadcasted_iota(jnp.int32, (BLK, 1), 0)
    hb = jnp.where(rowid < N, hb, 0.0)
    deg = deg2_ref[0, :, 0:1] + deg2_ref[1, :, 0:1]
    pos = deg > 0.0
    dis = jnp.where(pos, lax.rsqrt(jnp.where(pos, deg, 1.0)), 0.0)
    coe0 = _coe_dot(0, temp_ref)
    ub = dis * hb
    h_ref[...] = hb
    for p in range(PH):
        u_ref[p] = ub[:, p * FH:(p + 1) * FH]
    out0_ref[...] = (0.5 * coe0) * hb
    dis_ref[...] = dis


_mlp = pl.pallas_call(
    _mlp_body,
    grid=(GRID,),
    in_specs=[
        pl.BlockSpec((BLK, F), lambda b: (b, 0)),
        pl.BlockSpec((F, F), lambda b: (0, 0)),
        pl.BlockSpec((1, F), lambda b: (0, 0)),
        pl.BlockSpec((F, F), lambda b: (0, 0)),
        pl.BlockSpec((1, F), lambda b: (0, 0)),
        pl.BlockSpec((NC, BLK, DW), lambda b: (0, b, 0)),
        pl.BlockSpec((1, 128), lambda b: (0, 0)),
    ],
    out_specs=[
        pl.BlockSpec((BLK, F), lambda b: (b, 0)),
        pl.BlockSpec((PH, BLK, FH), lambda b: (0, b, 0)),
        pl.BlockSpec((BLK, F), lambda b: (b, 0)),
        pl.BlockSpec((BLK, 1), lambda b: (b, 0)),
    ],
    out_shape=[
        jax.ShapeDtypeStruct((NPAD, F), jnp.float32),
        jax.ShapeDtypeStruct((PH, NPAD, FH), jnp.float32),
        jax.ShapeDtypeStruct((NPAD, F), jnp.float32),
        jax.ShapeDtypeStruct((NPAD, 1), jnp.float32),
    ],
)


def _step_body(s_ref, dis_ref, tp_ref, oin_ref, temp_ref,
               tx_ref, out_ref, u_ref, *, alpha, beta, i):
    parts = [s_ref[0, p] + s_ref[1, p] for p in range(PH)]
    sblk = parts[0] if PH == 1 else jnp.concatenate(parts, axis=1)
    d = dis_ref[...]
    ci = _coe_dot(i, temp_ref)
    t = alpha * (d * sblk) + beta * tp_ref[...]
    ut = d * t
    tx_ref[...] = t
    out_ref[...] = oin_ref[...] + ci * t
    for p in range(PH):
        u_ref[p] = ut[:, p * FH:(p + 1) * FH]


def _make_step(i):
    alpha = -1.0 if i == 1 else -2.0
    beta = 0.0 if i == 1 else -1.0
    return pl.pallas_call(
        functools.partial(_step_body, alpha=alpha, beta=beta, i=i),
        grid=(GRID,),
        in_specs=[
            pl.BlockSpec((NC, PH, BLK, FH), lambda b: (0, 0, b, 0)),
            pl.BlockSpec((BLK, 1), lambda b: (b, 0)),
            pl.BlockSpec((BLK, F), lambda b: (b, 0)),
            pl.BlockSpec((BLK, F), lambda b: (b, 0)),
            pl.BlockSpec((1, 128), lambda b: (0, 0)),
        ],
        out_specs=[
            pl.BlockSpec((BLK, F), lambda b: (b, 0)),
            pl.BlockSpec((BLK, F), lambda b: (b, 0)),
            pl.BlockSpec((PH, BLK, FH), lambda b: (0, b, 0)),
        ],
        out_shape=[
            jax.ShapeDtypeStruct((NPAD, F), jnp.float32),
            jax.ShapeDtypeStruct((NPAD, F), jnp.float32),
            jax.ShapeDtypeStruct((PH, NPAD, FH), jnp.float32),
        ],
    )


_steps = {i: _make_step(i) for i in range(1, K + 1)}


def kernel(x, edge_index, edge_att, W1, b1, W2, b2, temp):
    del edge_att
    row = edge_index[0]
    col = edge_index[1]
    fill = N + jnp.arange(EPAD - E, dtype=jnp.int32) % (NPAD - N)
    srcs = jnp.concatenate([row, fill]).reshape(NW, CH, KB)
    dsts = jnp.concatenate([col, fill]).reshape(NW, CH, KB)
    packed = jnp.bitwise_or(jnp.left_shift(dsts, 16), srcs)
    x_pad = jnp.pad(x, ((0, NPAD - N), (0, 0)))
    temp_pad = jnp.zeros((1, 128), jnp.float32).at[0, : K + 1].set(temp)
    zrows = jnp.zeros((KB, FH), jnp.float32)
    zrows_d = jnp.zeros((KB, DW), jnp.float32)
    ones_d = jnp.ones((KB, DW), jnp.float32)

    deg2 = _sc_deg(srcs, ones_d, zrows_d)
    h, u, out, dis = _mlp(x_pad, W1, b1.reshape(1, F), W2, b2.reshape(1, F),
                          deg2, temp_pad)
    tx0 = h
    s_p = _sc_prop(u, packed, zrows)
    tx1, out, u = _steps[1](s_p, dis, tx0, out, temp_pad)
    for i in range(2, K + 1):
        s_p = _sc_prop(u, packed, zrows)
        tx2, out, u = _steps[i](s_p, dis, tx1, out, temp_pad)
        tx0, tx1 = tx1, tx2
    return out[:N]
